# Initial kernel scaffold; baseline (speedup 1.0000x reference)
import jax
import jax.numpy as jnp
from jax.experimental import pallas as pl


def kernel(hypotheses, references):
    raise NotImplementedError("write your pallas kernel here")


# speedup vs baseline: 6.9354x; 6.9354x over previous
import functools
import math

import jax
import jax.numpy as jnp
from jax.experimental import pallas as pl
from jax.experimental.pallas import tpu as pltpu

PAD = 0
RATE = 0.1


def _body(h_ref, r_ref, o_ref, rs_acc, gold_acc, h0_acc, *, nv, vb, v, k1, c):
    i = pl.program_id(0)
    j = pl.program_id(1)
    blk = h_ref[...]
    r = r_ref[...]

    @pl.when(j == 0)
    def _init():
        rs_acc[...] = jnp.zeros_like(rs_acc)
        gold_acc[...] = jnp.zeros_like(gold_acc)
        h0_acc[...] = blk[:, :, 0]

    @pl.when(jnp.logical_and(i == 0, j == 0))
    def _init_out():
        o_ref[...] = jnp.zeros_like(o_ref)

    rs_acc[...] += jnp.sum(blk, axis=2)
    ids = jax.lax.broadcasted_iota(jnp.int32, blk.shape, 2) + j * vb
    rr = jnp.transpose(r, (0, 2, 1))
    gold_acc[...] += jnp.sum(jnp.where(ids == rr, blk, 0.0), axis=2)

    @pl.when(j == nv - 1)
    def _fold():
        rowsum = rs_acc[...]
        gold = gold_acc[...]
        h0 = h0_acc[...]
        nonpad = (jnp.transpose(r, (0, 2, 1))[..., 0] != PAD)
        tail = jnp.where(nonpad, c * h0, c * math.log(c))
        loss_row = k1 - c * rowsum + (c - RATE) * gold + tail
        o_ref[...] += jnp.sum(loss_row).reshape(1, 1)


def kernel(hypotheses, references):
    B, S, V = hypotheses.shape
    c = (1.0 - RATE) / (V - 2)
    k1 = (V - 2) * c * math.log(c) + RATE * math.log(RATE)
    VB = 6400
    NV = V // VB
    refs = references.astype(jnp.int32).reshape(B, 1, S)

    out = pl.pallas_call(
        functools.partial(_body, nv=NV, vb=VB, v=V, k1=k1, c=c),
        grid=(B, NV),
        in_specs=[
            pl.BlockSpec((1, S, VB), lambda i, j: (i, 0, j)),
            pl.BlockSpec((1, 1, S), lambda i, j: (i, 0, 0)),
        ],
        out_specs=pl.BlockSpec((1, 1), lambda i, j: (0, 0)),
        out_shape=jax.ShapeDtypeStruct((1, 1), jnp.float32),
        scratch_shapes=[
            pltpu.VMEM((1, S), jnp.float32),
            pltpu.VMEM((1, S), jnp.float32),
            pltpu.VMEM((1, S), jnp.float32),
        ],
    )(hypotheses, refs)
    return out[0, 0]
